# baseline (speedup 1.0000x reference)
import jax
import jax.numpy as jnp
from jax import lax
from jax.experimental import pallas as pl
from jax.experimental.pallas import tpu as pltpu
from jax.experimental.pallas import tpu_sc as plsc

FEATURE_DIM = 100000
EMBEDDING_DIM = 64

NUM_CORES = 2
NUM_SUBCORES = 16
NUM_WORKERS = NUM_CORES * NUM_SUBCORES

BATCH = 4096
SEQ = 50
TOTAL = BATCH * SEQ
PER_WORKER = TOTAL // NUM_WORKERS
B_PER_WORKER = BATCH // NUM_WORKERS
B_PER_CHUNK = 16
CHUNK = B_PER_CHUNK * SEQ
NUM_CHUNKS = B_PER_WORKER // B_PER_CHUNK


def _gather_body(idx_hbm, table_hbm, out_hbm,
                 idx_v, rows0, rows1, g0, g1, s0, s1):
    wid = lax.axis_index("s") * NUM_CORES + lax.axis_index("c")
    base = wid * PER_WORKER
    base_b = wid * B_PER_WORKER

    rows_b = (rows0, rows1)
    gsem = (g0, g1)
    ssem = (s0, s1)
    gathers = [None, None]
    stores = [[], []]

    pltpu.sync_copy(idx_hbm.at[pl.ds(base, PER_WORKER)], idx_v)

    def start_stores(i):
        b = i % 2
        stores[b] = []
        for k in range(B_PER_CHUNK):
            stores[b].append(pltpu.async_copy(
                rows_b[b].at[pl.ds(k * SEQ, SEQ)],
                out_hbm.at[base_b + i * B_PER_CHUNK + k],
                ssem[b]))

    for i in range(NUM_CHUNKS):
        b = i % 2
        for st in stores[b]:
            st.wait()
        stores[b] = []
        gathers[b] = pltpu.async_copy(
            table_hbm.at[idx_v.at[pl.ds(i * CHUNK, CHUNK)]], rows_b[b],
            gsem[b])
        if i >= 1:
            gathers[1 - b].wait()
            start_stores(i - 1)

    last = (NUM_CHUNKS - 1) % 2
    gathers[last].wait()
    start_stores(NUM_CHUNKS - 1)
    for b in (1 - last, last):
        for st in stores[b]:
            st.wait()


@jax.jit
def _gather(idx, table):
    mesh = plsc.VectorSubcoreMesh(core_axis_name="c", subcore_axis_name="s",
                                  num_cores=NUM_CORES,
                                  num_subcores=NUM_SUBCORES)
    return pl.kernel(
        _gather_body,
        out_type=jax.ShapeDtypeStruct((BATCH, SEQ, EMBEDDING_DIM),
                                      jnp.float32),
        mesh=mesh,
        scratch_types=[
            pltpu.VMEM((PER_WORKER,), jnp.int32),
            pltpu.VMEM((CHUNK, EMBEDDING_DIM), jnp.float32),
            pltpu.VMEM((CHUNK, EMBEDDING_DIM), jnp.float32),
            pltpu.SemaphoreType.DMA,
            pltpu.SemaphoreType.DMA,
            pltpu.SemaphoreType.DMA,
            pltpu.SemaphoreType.DMA,
        ],
        compiler_params=pltpu.CompilerParams(use_tc_tiling_on_sc=False),
    )(idx, table)


def kernel(x, table):
    idx = x.reshape(-1).astype(jnp.int32)
    return _gather(idx, table)
